# baseline (speedup 1.0000x reference)
import functools

import jax
import jax.numpy as jnp
from jax import lax
from jax.experimental import pallas as pl
from jax.experimental.pallas import tpu as pltpu
from jax.experimental.pallas import tpu_sc as plsc

NC, NS = 2, 16
NW = NC * NS
CHUNK = 128
BLK = 2000


def _npad(n):
    return ((n + 16 + NS * 8 - 1) // (NS * 8)) * (NS * 8)


def _sc_mesh():
    return plsc.VectorSubcoreMesh(
        core_axis_name="c", subcore_axis_name="s",
        num_cores=NC, num_subcores=NS)


def _make_propagate(npad, n_chunks, grp, colsplit):
    d = 64
    assert n_chunks % grp == 0
    rows_per_tile = npad // NS

    @functools.partial(
        pl.kernel,
        out_type=jax.ShapeDtypeStruct((npad, 2 * d), jnp.float32),
        mesh=_sc_mesh(),
        compiler_params=pltpu.CompilerParams(use_tc_tiling_on_sc=False),
        scratch_types=[
            pltpu.VMEM((n_chunks, CHUNK), jnp.int32),
            pltpu.VMEM((n_chunks, CHUNK), jnp.int32),
            pltpu.VMEM((grp, CHUNK, d), jnp.float32),
            pltpu.VMEM_SHARED((npad, d), jnp.float32),
            pltpu.SemaphoreType.DMA,
            pltpu.SemaphoreType.DMA,
        ],
    )
    def prop(y2_hbm, src_hbm, dst_hbm, zeros_hbm, out_hbm,
             src_v, dst_v, rows_v, acc, sem, ssem):
        c = lax.axis_index("c")
        s = lax.axis_index("s")
        base = s * rows_per_tile
        pltpu.sync_copy(zeros_hbm.at[pl.ds(base, rows_per_tile)],
                        acc.at[pl.ds(base, rows_per_tile)])
        if colsplit:
            pltpu.sync_copy(src_hbm.at[c, s], src_v)
            pltpu.sync_copy(dst_hbm.at[s], dst_v)
        else:
            wid = c * NS + s
            pltpu.sync_copy(src_hbm.at[wid], src_v)
            pltpu.sync_copy(dst_hbm.at[wid], dst_v)
        plsc.subcore_barrier()

        def body(g, carry):
            ch = g * grp
            descs = [
                pltpu.async_copy(y2_hbm.at[src_v.at[ch + b]], rows_v.at[b],
                                 sem)
                for b in range(grp)
            ]
            sdescs = []
            for b in range(grp):
                descs[b].wait()
                sdescs.append(pltpu.async_copy(
                    rows_v.at[b], acc.at[dst_v.at[ch + b]], ssem, add=True))
            for sd in sdescs:
                sd.wait()
            return carry

        lax.fori_loop(0, n_chunks // grp, body, 0)
        plsc.subcore_barrier()
        pltpu.sync_copy(acc.at[pl.ds(base, rows_per_tile)],
                        out_hbm.at[pl.ds(base, rows_per_tile),
                                   pl.ds(pl.multiple_of(c * d, d), d)])

    return prop


def _make_degree(npad, n_chunks):
    d = 16
    rows_per_tile = npad // NS

    @functools.partial(
        pl.kernel,
        out_type=jax.ShapeDtypeStruct((npad, 128), jnp.float32),
        mesh=_sc_mesh(),
        compiler_params=pltpu.CompilerParams(use_tc_tiling_on_sc=False),
        scratch_types=[
            pltpu.VMEM((n_chunks, CHUNK), jnp.int32),
            pltpu.VMEM((CHUNK, d), jnp.float32),
            pltpu.VMEM_SHARED((npad, d), jnp.float32),
            pltpu.SemaphoreType.DMA,
        ],
    )
    def degk(dst_hbm, ones_hbm, zeros_hbm, out_hbm, dst_v, ones_v, acc, ssem):
        c = lax.axis_index("c")
        s = lax.axis_index("s")
        wid = c * NS + s
        base = s * rows_per_tile
        pltpu.sync_copy(zeros_hbm.at[pl.ds(base, rows_per_tile)],
                        acc.at[pl.ds(base, rows_per_tile)])
        pltpu.sync_copy(dst_hbm.at[wid], dst_v)
        pltpu.sync_copy(ones_hbm, ones_v)
        plsc.subcore_barrier()

        def body(g, carry):
            ch = g * 8
            sdescs = [
                pltpu.async_copy(ones_v, acc.at[dst_v.at[ch + b]], ssem,
                                 add=True)
                for b in range(8)
            ]
            for sd in sdescs:
                sd.wait()
            return carry

        lax.fori_loop(0, n_chunks // 8, body, 0)
        plsc.subcore_barrier()
        pltpu.sync_copy(acc.at[pl.ds(base, rows_per_tile)],
                        out_hbm.at[pl.ds(base, rows_per_tile),
                                   pl.ds(pl.multiple_of(c * 64, 64), d)])

    return degk


def _make_idx_body(n, e, cap, nw, nch):
    pad = cap - e

    def body(ei_ref, srco_ref, src2_ref, dstp_ref):
        ei = ei_ref[...]
        pad_i = lax.broadcasted_iota(jnp.int32, (pad,), 0)
        srcf = jnp.concatenate([ei[0], pad_i % n])
        dstf = jnp.concatenate([ei[1], n + (pad_i % 16)])
        so = 2 * srcf + 1
        srco_ref[...] = so.reshape(nw, nch, CHUNK)
        src2_ref[...] = jnp.stack([
            (2 * srcf).reshape(NS, 2 * nch, CHUNK),
            so.reshape(NS, 2 * nch, CHUNK),
        ])
        dstp_ref[...] = dstf.reshape(nw, nch, CHUNK)

    return body


def _tca_body(x_ref, w_ref, xw_ref):
    xw_ref[...] = jnp.dot(x_ref[...], w_ref[...],
                          preferred_element_type=jnp.float32)


def _tcc_body(scpk_ref, zz_ref, y0_ref, dinv_ref, b1_ref, w2_ref, hvp_ref):
    dc = dinv_ref[...][:, 0:1]
    zz = zz_ref[...]
    pp = (scpk_ref[...][:, 0:64] + scpk_ref[...][:, 64:128]
          + zz[:, 64:128]) * dc
    h = jnp.concatenate([y0_ref[...], zz[:, 0:64], pp], axis=1)
    h = jnp.maximum(h + b1_ref[...], 0.0)
    hw = jnp.dot(h, w2_ref[...], preferred_element_type=jnp.float32)
    zpad = jnp.zeros((hw.shape[0], 24), jnp.float32)
    hvp_ref[...] = jnp.concatenate(
        [hw[:, 0:40], zpad, hw[:, 40:80] * dc, zpad], axis=1)


def _tcd_body(sdpk_ref, hvp_ref, dinv_ref, b2_ref, out_ref):
    dc = dinv_ref[...][:, 0:1]
    hvp = hvp_ref[...]
    pv = (sdpk_ref[...][:, 0:40] + sdpk_ref[...][:, 64:104]
          + hvp[:, 64:104]) * dc
    logits = jnp.concatenate([hvp[:, 0:40], pv], axis=1) + b2_ref[...]
    m = jnp.max(logits, axis=1, keepdims=True)
    e = jnp.exp(logits - m)
    lse = jnp.log(jnp.sum(e, axis=1, keepdims=True))
    out_ref[...] = logits - m - lse


def _row_spec(d):
    return pl.BlockSpec((BLK, d), lambda i: (i, 0))


def _full_spec(r, c):
    return pl.BlockSpec((r, c), lambda i: (0, 0))


def _sds(r, c):
    return jax.ShapeDtypeStruct((r, c), jnp.float32)


def kernel(x, edge_index, W1_0, W1_1, W1_2, b1, W2_0, W2_1, b2):
    n = x.shape[0]
    e = edge_index.shape[1]
    npad = _npad(n)
    grid = (n + BLK - 1) // BLK

    ei32 = edge_index.astype(jnp.int32)
    per_tile = -(-e // NW)
    n_chunks = -(-per_tile // CHUNK)
    n_chunks = -(-n_chunks // 8) * 8
    cap = NW * n_chunks * CHUNK
    nch16 = 2 * n_chunks

    src_o32, src2, dst_p = pl.pallas_call(
        _make_idx_body(n, e, cap, NW, n_chunks),
        grid=(1,),
        in_specs=[_full_spec(2, e)],
        out_specs=[
            pl.BlockSpec((NW, n_chunks, CHUNK), lambda i: (0, 0, 0)),
            pl.BlockSpec((2, NS, nch16, CHUNK), lambda i: (0, 0, 0, 0)),
            pl.BlockSpec((NW, n_chunks, CHUNK), lambda i: (0, 0, 0)),
        ],
        out_shape=[
            jax.ShapeDtypeStruct((NW, n_chunks, CHUNK), jnp.int32),
            jax.ShapeDtypeStruct((2, NS, nch16, CHUNK), jnp.int32),
            jax.ShapeDtypeStruct((NW, n_chunks, CHUNK), jnp.int32),
        ],
    )(ei32)
    dst16 = dst_p.reshape(NS, nch16, CHUNK)

    ones16 = jnp.ones((CHUNK, 16), jnp.float32)
    zeros64 = jnp.zeros((npad, 64), jnp.float32)
    zeros16 = jnp.zeros((npad, 16), jnp.float32)

    w1cat = jnp.concatenate([W1_0, W1_1, W1_2], axis=1)
    w2cat = jnp.concatenate([W2_0, W2_1], axis=1)
    b1r = b1.reshape(1, -1)
    b2r = b2.reshape(1, -1)

    degpk = _make_degree(npad, n_chunks)(dst_p, ones16, zeros16)

    xw = pl.pallas_call(
        _tca_body,
        grid=(grid,),
        in_specs=[_row_spec(128), _full_spec(128, 192)],
        out_specs=_row_spec(192),
        out_shape=_sds(n, 192),
    )(x, w1cat)

    deg = degpk[:n, 0:1] + degpk[:n, 64:65] + 1.0
    dinv16 = jnp.broadcast_to(1.0 / jnp.sqrt(deg), (n, 16))
    ys = xw[:, 64:192] * dinv16[:, 0:1]
    ysv = ys.reshape(2 * n, 64)
    y0 = xw[:, 0:64]

    sb = _make_propagate(npad, nch16, 5, True)(ysv, src2, dst16, zeros64)

    dc = dinv16[:, 0:1]
    zz = jnp.concatenate([
        (sb[:n, 0:64] + ys[:, 0:64]) * dc,
        (sb[:n, 64:128] + ys[:, 64:128]) * (dc * dc),
    ], axis=1)

    scpk = _make_propagate(npad, n_chunks, 8, False)(
        zz.reshape(2 * n, 64), src_o32, dst_p, zeros64)

    hvp = pl.pallas_call(
        _tcc_body,
        grid=(grid,),
        in_specs=[_row_spec(128), _row_spec(128), _row_spec(64), _row_spec(16),
                  _full_spec(1, 192), _full_spec(192, 80)],
        out_specs=_row_spec(128),
        out_shape=_sds(n, 128),
    )(scpk, zz, y0, dinv16, b1r, w2cat)

    sdpk = _make_propagate(npad, n_chunks, 8, False)(
        hvp.reshape(2 * n, 64), src_o32, dst_p, zeros64)

    out = pl.pallas_call(
        _tcd_body,
        grid=(grid,),
        in_specs=[_row_spec(128), _row_spec(128), _row_spec(16),
                  _full_spec(1, 80)],
        out_specs=_row_spec(80),
        out_shape=_sds(n, 80),
    )(sdpk, hvp, dinv16, b2r)

    return out


# speedup vs baseline: 1.0359x; 1.0359x over previous
import functools

import jax
import jax.numpy as jnp
from jax import lax
from jax.experimental import pallas as pl
from jax.experimental.pallas import tpu as pltpu
from jax.experimental.pallas import tpu_sc as plsc

NC, NS = 2, 16
NW = NC * NS
CHUNK = 128
BLK = 5000


def _npad(n):
    return ((n + 16 + NS * 8 - 1) // (NS * 8)) * (NS * 8)


def _sc_mesh():
    return plsc.VectorSubcoreMesh(
        core_axis_name="c", subcore_axis_name="s",
        num_cores=NC, num_subcores=NS)


def _make_propagate(npad, n_chunks, grp, colsplit):
    d = 64
    assert n_chunks % grp == 0
    rows_per_tile = npad // NS

    @functools.partial(
        pl.kernel,
        out_type=jax.ShapeDtypeStruct((npad, 2 * d), jnp.float32),
        mesh=_sc_mesh(),
        compiler_params=pltpu.CompilerParams(use_tc_tiling_on_sc=False),
        scratch_types=[
            pltpu.VMEM((n_chunks, CHUNK), jnp.int32),
            pltpu.VMEM((n_chunks, CHUNK), jnp.int32),
            pltpu.VMEM((grp, CHUNK, d), jnp.float32),
            pltpu.VMEM_SHARED((npad, d), jnp.float32),
            pltpu.SemaphoreType.DMA,
            pltpu.SemaphoreType.DMA,
        ],
    )
    def prop(y2_hbm, src_hbm, dst_hbm, zeros_hbm, out_hbm,
             src_v, dst_v, rows_v, acc, sem, ssem):
        c = lax.axis_index("c")
        s = lax.axis_index("s")
        base = s * rows_per_tile
        pltpu.sync_copy(zeros_hbm.at[pl.ds(base, rows_per_tile)],
                        acc.at[pl.ds(base, rows_per_tile)])
        if colsplit:
            pltpu.sync_copy(src_hbm.at[c, s], src_v)
            pltpu.sync_copy(dst_hbm.at[s], dst_v)
        else:
            wid = c * NS + s
            pltpu.sync_copy(src_hbm.at[wid], src_v)
            pltpu.sync_copy(dst_hbm.at[wid], dst_v)
        plsc.subcore_barrier()

        def body(g, carry):
            ch = g * grp
            descs = [
                pltpu.async_copy(y2_hbm.at[src_v.at[ch + b]], rows_v.at[b],
                                 sem)
                for b in range(grp)
            ]
            sdescs = []
            for b in range(grp):
                descs[b].wait()
                sdescs.append(pltpu.async_copy(
                    rows_v.at[b], acc.at[dst_v.at[ch + b]], ssem, add=True))
            for sd in sdescs:
                sd.wait()
            return carry

        lax.fori_loop(0, n_chunks // grp, body, 0)
        plsc.subcore_barrier()
        pltpu.sync_copy(acc.at[pl.ds(base, rows_per_tile)],
                        out_hbm.at[pl.ds(base, rows_per_tile),
                                   pl.ds(pl.multiple_of(c * d, d), d)])

    return prop


def _make_degree(npad, n_chunks):
    d = 16
    rows_per_tile = npad // NS

    @functools.partial(
        pl.kernel,
        out_type=jax.ShapeDtypeStruct((npad, 128), jnp.float32),
        mesh=_sc_mesh(),
        compiler_params=pltpu.CompilerParams(use_tc_tiling_on_sc=False),
        scratch_types=[
            pltpu.VMEM((n_chunks, CHUNK), jnp.int32),
            pltpu.VMEM((CHUNK, d), jnp.float32),
            pltpu.VMEM_SHARED((npad, d), jnp.float32),
            pltpu.SemaphoreType.DMA,
        ],
    )
    def degk(dst_hbm, ones_hbm, zeros_hbm, out_hbm, dst_v, ones_v, acc, ssem):
        c = lax.axis_index("c")
        s = lax.axis_index("s")
        wid = c * NS + s
        base = s * rows_per_tile
        pltpu.sync_copy(zeros_hbm.at[pl.ds(base, rows_per_tile)],
                        acc.at[pl.ds(base, rows_per_tile)])
        pltpu.sync_copy(dst_hbm.at[wid], dst_v)
        pltpu.sync_copy(ones_hbm, ones_v)
        plsc.subcore_barrier()

        def body(g, carry):
            ch = g * 8
            sdescs = [
                pltpu.async_copy(ones_v, acc.at[dst_v.at[ch + b]], ssem,
                                 add=True)
                for b in range(8)
            ]
            for sd in sdescs:
                sd.wait()
            return carry

        lax.fori_loop(0, n_chunks // 8, body, 0)
        plsc.subcore_barrier()
        pltpu.sync_copy(acc.at[pl.ds(base, rows_per_tile)],
                        out_hbm.at[pl.ds(base, rows_per_tile),
                                   pl.ds(pl.multiple_of(c * 64, 64), d)])

    return degk


def _make_idx_body(n, e, cap, nw, nch):
    pad = cap - e

    def body(ei_ref, srco_ref, src2_ref, dstp_ref):
        ei = ei_ref[...]
        pad_i = lax.broadcasted_iota(jnp.int32, (pad,), 0)
        srcf = jnp.concatenate([ei[0], pad_i % n])
        dstf = jnp.concatenate([ei[1], n + (pad_i % 16)])
        so = 2 * srcf + 1
        srco_ref[...] = so.reshape(nw, nch, CHUNK)
        src2_ref[...] = jnp.stack([
            (2 * srcf).reshape(NS, 2 * nch, CHUNK),
            so.reshape(NS, 2 * nch, CHUNK),
        ])
        dstp_ref[...] = dstf.reshape(nw, nch, CHUNK)

    return body


def _tca_body(x_ref, w_ref, degpk_ref, y0_ref, ys_ref, dinv_ref):
    xw = jnp.dot(x_ref[...], w_ref[...], preferred_element_type=jnp.float32)
    deg = degpk_ref[...][:, 0:1] + degpk_ref[...][:, 64:65] + 1.0
    dc = 1.0 / jnp.sqrt(deg)
    y0_ref[...] = xw[:, 0:64]
    ys_ref[...] = xw[:, 64:192] * dc
    dinv_ref[...] = jnp.broadcast_to(dc, (dc.shape[0], 16))


def _tcc_body(scpk_ref, zz_ref, y0_ref, dinv_ref, b1_ref, w2_ref, hvp_ref):
    dc = dinv_ref[...][:, 0:1]
    zz = zz_ref[...]
    pp = (scpk_ref[...][:, 0:64] + scpk_ref[...][:, 64:128]
          + zz[:, 64:128]) * dc
    h = jnp.concatenate([y0_ref[...], zz[:, 0:64], pp], axis=1)
    h = jnp.maximum(h + b1_ref[...], 0.0)
    hw = jnp.dot(h, w2_ref[...], preferred_element_type=jnp.float32)
    zpad = jnp.zeros((hw.shape[0], 24), jnp.float32)
    hvp_ref[...] = jnp.concatenate(
        [hw[:, 0:40], zpad, hw[:, 40:80] * dc, zpad], axis=1)


def _tcd_body(sdpk_ref, hvp_ref, dinv_ref, b2_ref, out_ref):
    dc = dinv_ref[...][:, 0:1]
    hvp = hvp_ref[...]
    pv = (sdpk_ref[...][:, 0:40] + sdpk_ref[...][:, 64:104]
          + hvp[:, 64:104]) * dc
    logits = jnp.concatenate([hvp[:, 0:40], pv], axis=1) + b2_ref[...]
    m = jnp.max(logits, axis=1, keepdims=True)
    e = jnp.exp(logits - m)
    lse = jnp.log(jnp.sum(e, axis=1, keepdims=True))
    out_ref[...] = logits - m - lse


def _row_spec(d):
    return pl.BlockSpec((BLK, d), lambda i: (i, 0))


def _full_spec(r, c):
    return pl.BlockSpec((r, c), lambda i: (0, 0))


def _sds(r, c):
    return jax.ShapeDtypeStruct((r, c), jnp.float32)


def kernel(x, edge_index, W1_0, W1_1, W1_2, b1, W2_0, W2_1, b2):
    n = x.shape[0]
    e = edge_index.shape[1]
    npad = _npad(n)
    grid = (n + BLK - 1) // BLK

    ei32 = edge_index.astype(jnp.int32)
    per_tile = -(-e // NW)
    n_chunks = -(-per_tile // CHUNK)
    n_chunks = -(-n_chunks // 8) * 8
    cap = NW * n_chunks * CHUNK
    nch16 = 2 * n_chunks

    src_o32, src2, dst_p = pl.pallas_call(
        _make_idx_body(n, e, cap, NW, n_chunks),
        grid=(1,),
        in_specs=[_full_spec(2, e)],
        out_specs=[
            pl.BlockSpec((NW, n_chunks, CHUNK), lambda i: (0, 0, 0)),
            pl.BlockSpec((2, NS, nch16, CHUNK), lambda i: (0, 0, 0, 0)),
            pl.BlockSpec((NW, n_chunks, CHUNK), lambda i: (0, 0, 0)),
        ],
        out_shape=[
            jax.ShapeDtypeStruct((NW, n_chunks, CHUNK), jnp.int32),
            jax.ShapeDtypeStruct((2, NS, nch16, CHUNK), jnp.int32),
            jax.ShapeDtypeStruct((NW, n_chunks, CHUNK), jnp.int32),
        ],
    )(ei32)
    dst16 = dst_p.reshape(NS, nch16, CHUNK)

    ones16 = jnp.ones((CHUNK, 16), jnp.float32)
    zeros64 = jnp.zeros((npad, 64), jnp.float32)
    zeros16 = jnp.zeros((npad, 16), jnp.float32)

    w1cat = jnp.concatenate([W1_0, W1_1, W1_2], axis=1)
    w2cat = jnp.concatenate([W2_0, W2_1], axis=1)
    b1r = b1.reshape(1, -1)
    b2r = b2.reshape(1, -1)

    degpk = _make_degree(npad, n_chunks)(dst_p, ones16, zeros16)

    y0, ys, dinv16 = pl.pallas_call(
        _tca_body,
        grid=(grid,),
        in_specs=[_row_spec(128), _full_spec(128, 192), _row_spec(128)],
        out_specs=[_row_spec(64), _row_spec(128), _row_spec(16)],
        out_shape=[_sds(n, 64), _sds(n, 128), _sds(n, 16)],
    )(x, w1cat, degpk)
    ysv = ys.reshape(2 * n, 64)

    sb = _make_propagate(npad, nch16, 5, True)(ysv, src2, dst16, zeros64)

    dc = dinv16[:, 0:1]
    zz = jnp.concatenate([
        (sb[:n, 0:64] + ys[:, 0:64]) * dc,
        (sb[:n, 64:128] + ys[:, 64:128]) * (dc * dc),
    ], axis=1)

    scpk = _make_propagate(npad, n_chunks, 8, False)(
        zz.reshape(2 * n, 64), src_o32, dst_p, zeros64)

    hvp = pl.pallas_call(
        _tcc_body,
        grid=(grid,),
        in_specs=[_row_spec(128), _row_spec(128), _row_spec(64), _row_spec(16),
                  _full_spec(1, 192), _full_spec(192, 80)],
        out_specs=_row_spec(128),
        out_shape=_sds(n, 128),
    )(scpk, zz, y0, dinv16, b1r, w2cat)

    sdpk = _make_propagate(npad, n_chunks, 8, False)(
        hvp.reshape(2 * n, 64), src_o32, dst_p, zeros64)

    out = pl.pallas_call(
        _tcd_body,
        grid=(grid,),
        in_specs=[_row_spec(128), _row_spec(128), _row_spec(16),
                  _full_spec(1, 80)],
        out_specs=_row_spec(80),
        out_shape=_sds(n, 80),
    )(sdpk, hvp, dinv16, b2r)

    return out
